# baseline (speedup 1.0000x reference)
import functools

import jax
import jax.numpy as jnp
from jax import lax
from jax.experimental import pallas as pl
from jax.experimental.pallas import tpu as pltpu
from jax.experimental.pallas import tpu_sc as plsc

_CHUNK = 128
_B_TC = 8192
_BB = 1024


def _sc_gather_fn(V, B, D, nc, ns):
    nw = nc * ns
    b_per_w = B // nw
    n_chunks = b_per_w // _CHUNK
    n_stagers = 5
    v_per_stager = V // n_stagers
    mesh = plsc.VectorSubcoreMesh(core_axis_name="c", subcore_axis_name="s")

    @functools.partial(
        pl.kernel,
        mesh=mesh,
        out_type=jax.ShapeDtypeStruct((B, D), jnp.float32),
        scratch_types=[
            pltpu.VMEM((n_chunks, _CHUNK), jnp.int32),
            pltpu.VMEM((n_chunks, _CHUNK, D), jnp.float32),
            pltpu.VMEM_SHARED((V, D), jnp.float32),
            pltpu.SemaphoreType.DMA,
            pltpu.SemaphoreType.DMA,
        ],
    )
    def k(table_hbm, idx_hbm, out_hbm, idx_v, rows_v, table_sp, sem_g, sem_w):
        cid = lax.axis_index("c")
        sid = lax.axis_index("s")
        wid = sid * nc + cid
        base = wid * b_per_w

        @pl.when(sid < n_stagers)
        def _():
            r0 = sid * v_per_stager
            pltpu.sync_copy(
                table_hbm.at[pl.ds(r0, v_per_stager)],
                table_sp.at[pl.ds(r0, v_per_stager)],
            )

        pltpu.sync_copy(idx_hbm.at[wid], idx_v)
        plsc.subcore_barrier()

        gathers = [
            pltpu.async_copy(table_sp.at[idx_v.at[i]], rows_v.at[i], sem_g)
            for i in range(n_chunks)
        ]
        writes = []
        for i in range(n_chunks):
            gathers[i].wait()
            writes.append(
                pltpu.async_copy(
                    rows_v.at[i],
                    out_hbm.at[pl.ds(base + i * _CHUNK, _CHUNK)],
                    sem_w,
                )
            )
        for w in writes:
            w.wait()

    return k


def _tc_gather_fn(V, D, Bt):
    n_steps = Bt // _BB

    def body(idx_ref, table_ref, out_ref):
        iot = lax.broadcasted_iota(jnp.int32, (V, _BB), 0)
        oh = (iot == idx_ref[0]).astype(jnp.bfloat16)
        out_ref[...] = lax.dot_general(
            oh,
            table_ref[...],
            (((0,), (0,)), ((), ())),
            preferred_element_type=jnp.float32,
        )

    return pl.pallas_call(
        body,
        grid=(n_steps,),
        in_specs=[
            pl.BlockSpec((1, 1, _BB), lambda i: (i, 0, 0)),
            pl.BlockSpec((V, D), lambda i: (0, 0)),
        ],
        out_specs=pl.BlockSpec((_BB, D), lambda i: (i, 0)),
        out_shape=jax.ShapeDtypeStruct((Bt, D), jnp.float32),
    )


def kernel(seq, state_idx, embedding_table):
    V, D = embedding_table.shape
    B = state_idx.shape[0]
    info = plsc.get_sparse_core_info()
    nc, ns = info.num_cores, info.num_subcores
    b_sc = B - _B_TC
    idx_sc = state_idx[:b_sc].reshape(nc * ns, b_sc // (nc * ns) // _CHUNK, _CHUNK)
    idx_tc = state_idx[b_sc:].reshape(_B_TC // _BB, 1, _BB)
    table_bf = embedding_table.astype(jnp.bfloat16)
    sc_out = _sc_gather_fn(V, b_sc, D, nc, ns)(embedding_table, idx_sc)
    tc_out = _tc_gather_fn(V, D, _B_TC)(idx_tc, table_bf)
    return jnp.concatenate([sc_out, tc_out], axis=0)


# speedup vs baseline: 1.3345x; 1.3345x over previous
import functools

import jax
import jax.numpy as jnp
from jax import lax
from jax.experimental import pallas as pl
from jax.experimental.pallas import tpu as pltpu
from jax.experimental.pallas import tpu_sc as plsc

_CHUNK = 128


def _gather_fn(V, B, D, nc, ns):
    nw = nc * ns
    b_per_w = B // nw
    n_chunks = b_per_w // _CHUNK
    n_stagers = 5
    v_per_stager = V // n_stagers
    mesh = plsc.VectorSubcoreMesh(core_axis_name="c", subcore_axis_name="s")

    @functools.partial(
        pl.kernel,
        mesh=mesh,
        out_type=jax.ShapeDtypeStruct((B, D), jnp.float32),
        scratch_types=[
            pltpu.VMEM((n_chunks, _CHUNK), jnp.int32),
            pltpu.VMEM((n_chunks, _CHUNK, D), jnp.float32),
            pltpu.VMEM_SHARED((V, D), jnp.float32),
            pltpu.SemaphoreType.DMA,
            pltpu.SemaphoreType.DMA,
            pltpu.SemaphoreType.DMA,
        ],
    )
    def k(table_hbm, idx_hbm, out_hbm, idx_v, rows_v, table_sp,
          sem_h, sem_g, sem_w):
        cid = lax.axis_index("c")
        sid = lax.axis_index("s")
        wid = sid * nc + cid
        base = wid * b_per_w

        pltpu.sync_copy(idx_hbm.at[wid], idx_v)
        g0 = pltpu.async_copy(table_hbm.at[idx_v.at[0]], rows_v.at[0], sem_h)

        @pl.when(sid < n_stagers)
        def _():
            r0 = sid * v_per_stager
            pltpu.sync_copy(
                table_hbm.at[pl.ds(r0, v_per_stager)],
                table_sp.at[pl.ds(r0, v_per_stager)],
            )

        g0.wait()
        writes = [
            pltpu.async_copy(
                rows_v.at[0], out_hbm.at[pl.ds(base, _CHUNK)], sem_w
            )
        ]
        plsc.subcore_barrier()

        gathers = [None] + [
            pltpu.async_copy(table_sp.at[idx_v.at[i]], rows_v.at[i], sem_g)
            for i in range(1, n_chunks)
        ]
        for i in range(1, n_chunks):
            gathers[i].wait()
            writes.append(
                pltpu.async_copy(
                    rows_v.at[i],
                    out_hbm.at[pl.ds(base + i * _CHUNK, _CHUNK)],
                    sem_w,
                )
            )
        for w in writes:
            w.wait()

    return k


def kernel(seq, state_idx, embedding_table):
    V, D = embedding_table.shape
    B = state_idx.shape[0]
    info = plsc.get_sparse_core_info()
    nc, ns = info.num_cores, info.num_subcores
    idx = state_idx.reshape(nc * ns, B // (nc * ns) // _CHUNK, _CHUNK)
    return _gather_fn(V, B, D, nc, ns)(embedding_table, idx)
